# Initial kernel scaffold; baseline (speedup 1.0000x reference)
import jax
import jax.numpy as jnp
from jax.experimental import pallas as pl


def kernel(alpha, sigma_seed, inv_features, edge_index, loop_indicators, W_equ_in, b_equ_in, W_inv_in, b_inv_in, We, Wi, bi, Wc, H0, b0, H1, b1, H2, b2):
    raise NotImplementedError("write your pallas kernel here")


# speedup vs baseline: 1.0014x; 1.0014x over previous
import jax
import jax.numpy as jnp
from jax.experimental import pallas as pl

N_NODES = 10000
N_LOOPS = 8
EQU = 32
INV = 16
HID = 64
N_LAYERS = 4


def _head_pallas(pooled, H0, b0, H1, b1, H2, b2):
    def body(p_ref, h0_ref, b0_ref, h1_ref, b1_ref, h2_ref, b2_ref, o_ref):
        h = jax.nn.gelu(p_ref[...] @ h0_ref[...] + b0_ref[...][None, :])
        h = jax.nn.gelu(h @ h1_ref[...] + b1_ref[...][None, :])
        o_ref[...] = h @ h2_ref[...] + b2_ref[...][None, :]

    out = pl.pallas_call(
        body,
        out_shape=jax.ShapeDtypeStruct((N_LOOPS, 1), jnp.float32),
    )(pooled, H0, b0, H1, b1, H2, b2)
    return out[:, 0]


def kernel(alpha, sigma_seed, inv_features, edge_index, loop_indicators,
           W_equ_in, b_equ_in, W_inv_in, b_inv_in, We, Wi, bi, Wc,
           H0, b0, H1, b1, H2, b2):
    src = edge_index[0]
    dst = edge_index[1]
    Xi0 = inv_features @ W_inv_in + b_inv_in

    pooled_all = []
    sigma = sigma_seed
    masks = loop_indicators.astype(jnp.float32)
    for i in range(N_LOOPS):
        Xe = sigma[:, None] * W_equ_in[0][None, :] + b_equ_in
        Xi = Xi0
        for l in range(N_LAYERS):
            node_e = jnp.zeros((N_NODES, EQU), jnp.float32).at[src].add(Xe).at[dst].add(-Xe)
            msg_e = node_e[src] - node_e[dst]
            Xe = Xe + jnp.tanh(msg_e @ We[l])
            node_i = jnp.zeros((N_NODES, INV), jnp.float32).at[src].add(Xi).at[dst].add(Xi)
            msg_i = node_i[src] + node_i[dst]
            Xi = jax.nn.gelu(Xi + msg_i @ Wi[l] + bi[l] + (Xe ** 2) @ Wc[l])
        mask = masks[i]
        n_edges_in_loop = jnp.maximum(mask.sum(), 1.0)
        pooled_e = (Xe * mask[:, None]).sum(axis=0) / n_edges_in_loop
        pooled_i = (Xi * mask[:, None]).sum(axis=0) / n_edges_in_loop
        pooled_all.append(jnp.concatenate([pooled_e, pooled_i]))
        a_i = alpha[i].astype(jnp.float32)
        sigma = jnp.where(a_i > 0.5, sigma * (1.0 - 2.0 * mask), sigma)

    pooled = jnp.stack(pooled_all)
    logits = _head_pallas(pooled, H0, b0, H1, b1, H2, b2)
    p = jnp.clip(jax.nn.sigmoid(logits), 1e-06, 1.0 - 1e-06)
    a = alpha.astype(jnp.float32)
    log_prob = jnp.sum(a * jnp.log(p) + (1.0 - a) * jnp.log(1.0 - p))
    return log_prob


# speedup vs baseline: 6.2370x; 6.2282x over previous
import functools

import jax
import jax.numpy as jnp
from jax import lax
from jax.experimental import pallas as pl
from jax.experimental.pallas import tpu as pltpu
from jax.experimental.pallas import tpu_sc as plsc

N_NODES = 10000
N_EDGES = 320000
N_LOOPS = 8
EQU = 32
INV = 16
HID = 64
N_LAYERS = 4

HALF = 4
WE = HALF * EQU
WI = HALF * INV

NS = 16
TILE_E = N_EDGES // NS
CH = 80
NCH = TILE_E // CH
NPT = N_NODES // NS

BE = 2000
NB = N_EDGES // BE

_SC_PARAMS = pltpu.CompilerParams(use_tc_tiling_on_sc=False)


def _sc_scratch(width):
    return [
        pltpu.VMEM_SHARED((N_NODES, width), jnp.float32),
        pltpu.VMEM((CH,), jnp.int32),
        pltpu.VMEM((CH,), jnp.int32),
        pltpu.VMEM((CH, width), jnp.float32),
        pltpu.VMEM((CH, width), jnp.float32),
    ]


def _sc_zero_node(z_h, node):
    s = lax.axis_index("s")
    nb = s * NPT
    pltpu.sync_copy(z_h.at[pl.ds(nb, NPT)], node.at[pl.ds(nb, NPT)])
    plsc.subcore_barrier()


def _sc_gather_phase(c, src_h, dst_h, node, idx_s, idx_d, abuf, bbuf,
                     gs0, gd0, gs1, gd1):
    base0 = lax.axis_index("s") * TILE_E

    def gather_half(gs, gd):
        @pl.loop(0, NCH)
        def _(kk):
            base = base0 + kk * CH
            pltpu.sync_copy(src_h.at[pl.ds(base, CH)], idx_s)
            pltpu.sync_copy(dst_h.at[pl.ds(base, CH)], idx_d)
            pltpu.sync_copy(node.at[idx_s], abuf)
            pltpu.sync_copy(node.at[idx_d], bbuf)
            pltpu.sync_copy(abuf, gs.at[pl.ds(base, CH)])
            pltpu.sync_copy(bbuf, gd.at[pl.ds(base, CH)])

    @pl.when(c == 0)
    def _():
        gather_half(gs0, gd0)

    @pl.when(c == 1)
    def _():
        gather_half(gs1, gd1)


def _make_sc_e():
    mesh = plsc.VectorSubcoreMesh(core_axis_name="c", subcore_axis_name="s")
    out_t = [jax.ShapeDtypeStruct((N_EDGES, WE), jnp.float32)] * 4

    @functools.partial(
        pl.kernel,
        out_type=out_t,
        mesh=mesh,
        compiler_params=_SC_PARAMS,
        scratch_types=_sc_scratch(WE),
    )
    def k(a0_h, b0_h, a1_h, b1_h, src_h, dst_h, z_h, gs0, gd0, gs1, gd1,
          node, idx_s, idx_d, abuf, bbuf):
        c = lax.axis_index("c")
        _sc_zero_node(z_h, node)
        base0 = lax.axis_index("s") * TILE_E

        def scatter_half(a_h, b_h):
            @pl.loop(0, NCH)
            def _(kk):
                base = base0 + kk * CH
                pltpu.sync_copy(src_h.at[pl.ds(base, CH)], idx_s)
                pltpu.sync_copy(dst_h.at[pl.ds(base, CH)], idx_d)
                pltpu.sync_copy(a_h.at[pl.ds(base, CH)], abuf)
                pltpu.sync_copy(b_h.at[pl.ds(base, CH)], bbuf)
                pltpu.sync_copy(abuf, node.at[idx_s], add=True)
                pltpu.sync_copy(bbuf, node.at[idx_d], add=True)

        @pl.when(c == 0)
        def _():
            scatter_half(a0_h, b0_h)

        @pl.when(c == 1)
        def _():
            scatter_half(a1_h, b1_h)

        plsc.subcore_barrier()
        _sc_gather_phase(c, src_h, dst_h, node, idx_s, idx_d, abuf, bbuf,
                         gs0, gd0, gs1, gd1)

    return k


def _make_sc_i():
    mesh = plsc.VectorSubcoreMesh(core_axis_name="c", subcore_axis_name="s")
    out_t = [jax.ShapeDtypeStruct((N_EDGES, WI), jnp.float32)] * 4

    @functools.partial(
        pl.kernel,
        out_type=out_t,
        mesh=mesh,
        compiler_params=_SC_PARAMS,
        scratch_types=_sc_scratch(WI),
    )
    def k(x0_h, x1_h, src_h, dst_h, z_h, gs0, gd0, gs1, gd1,
          node, idx_s, idx_d, abuf, bbuf):
        c = lax.axis_index("c")
        _sc_zero_node(z_h, node)
        base0 = lax.axis_index("s") * TILE_E

        def scatter_half(x_h):
            @pl.loop(0, NCH)
            def _(kk):
                base = base0 + kk * CH
                pltpu.sync_copy(src_h.at[pl.ds(base, CH)], idx_s)
                pltpu.sync_copy(dst_h.at[pl.ds(base, CH)], idx_d)
                pltpu.sync_copy(x_h.at[pl.ds(base, CH)], abuf)
                pltpu.sync_copy(abuf, node.at[idx_s], add=True)
                pltpu.sync_copy(abuf, node.at[idx_d], add=True)

        @pl.when(c == 0)
        def _():
            scatter_half(x0_h)

        @pl.when(c == 1)
        def _():
            scatter_half(x1_h)

        plsc.subcore_barrier()
        _sc_gather_phase(c, src_h, dst_h, node, idx_s, idx_d, abuf, bbuf,
                         gs0, gd0, gs1, gd1)

    return k


def _init_body(mask_ref, sig_ref, inv_ref, alpha_ref, wequ_ref, bequ_ref,
               winv_ref, binv_ref, oae0, obe0, oae1, obe1, oxi0, oxi1):
    m = mask_ref[...]
    afl = alpha_ref[...]
    flip = 1.0 - 2.0 * m * afl
    inv = inv_ref[...]
    xi = (inv[:, 0:1] * winv_ref[...][0:1, :]
          + inv[:, 1:2] * winv_ref[...][1:2, :]
          + binv_ref[...])
    w_equ = wequ_ref[...]
    b_equ = bequ_ref[...]
    cur = sig_ref[...]
    xes = []
    for j in range(N_LOOPS):
        xes.append(cur * w_equ + b_equ)
        cur = cur * flip[:, j:j + 1]
    oae0[...] = jnp.concatenate(xes[0:4], axis=1)
    obe0[...] = jnp.concatenate([-x for x in xes[0:4]], axis=1)
    oae1[...] = jnp.concatenate(xes[4:8], axis=1)
    obe1[...] = jnp.concatenate([-x for x in xes[4:8]], axis=1)
    oxi0[...] = jnp.concatenate([xi] * HALF, axis=1)
    oxi1[...] = jnp.concatenate([xi] * HALF, axis=1)


def _init_call(masks_t, sig_seed, inv_features, alpha_row, w_equ, b_equ,
               w_inv, b_inv):
    blk = lambda r, c: pl.BlockSpec((r, c), lambda i: (i, 0))
    full = lambda r, c: pl.BlockSpec((r, c), lambda i: (0, 0))
    oshape = ([jax.ShapeDtypeStruct((N_EDGES, WE), jnp.float32)] * 4
              + [jax.ShapeDtypeStruct((N_EDGES, WI), jnp.float32)] * 2)
    return pl.pallas_call(
        _init_body,
        grid=(NB,),
        in_specs=[blk(BE, N_LOOPS), blk(BE, 1), blk(BE, 2), full(1, N_LOOPS),
                  full(1, EQU), full(1, EQU), full(2, INV), full(1, INV)],
        out_specs=[blk(BE, WE)] * 4 + [blk(BE, WI)] * 2,
        out_shape=oshape,
    )(masks_t, sig_seed, inv_features, alpha_row, w_equ, b_equ, w_inv, b_inv)


def _tce_body(ae0_ref, ae1_ref, gs0_ref, gd0_ref, gs1_ref, gd1_ref, we_ref,
              oa0, ob0, oa1, ob1):
    we = we_ref[...]

    def half(a_ref, gs_ref, gd_ref, oa, ob):
        msg = gs_ref[...] - gd_ref[...]
        xe = a_ref[...] + jnp.tanh(
            jnp.dot(msg, we, preferred_element_type=jnp.float32))
        oa[...] = xe
        ob[...] = -xe

    half(ae0_ref, gs0_ref, gd0_ref, oa0, ob0)
    half(ae1_ref, gs1_ref, gd1_ref, oa1, ob1)


def _tce_call(ae0, ae1, gs0, gd0, gs1, gd1, we_bd):
    blk = lambda: pl.BlockSpec((BE, WE), lambda i: (i, 0))
    full = pl.BlockSpec((WE, WE), lambda i: (0, 0))
    return pl.pallas_call(
        _tce_body,
        grid=(NB,),
        in_specs=[blk()] * 6 + [full],
        out_specs=[blk()] * 4,
        out_shape=[jax.ShapeDtypeStruct((N_EDGES, WE), jnp.float32)] * 4,
    )(ae0, ae1, gs0, gd0, gs1, gd1, we_bd)


def _tci_body(xi0_ref, xi1_ref, gs0_ref, gd0_ref, gs1_ref, gd1_ref,
              ae0_ref, ae1_ref, wi_ref, wc_ref, bi_ref, oxi0, oxi1):
    wi = wi_ref[...]
    wc = wc_ref[...]
    bi_t = bi_ref[...]

    def half(xi_ref, gs_ref, gd_ref, ae_ref, o):
        msg = gs_ref[...] + gd_ref[...]
        xe = ae_ref[...]
        pre = (xi_ref[...]
               + jnp.dot(msg, wi, preferred_element_type=jnp.float32)
               + bi_t
               + jnp.dot(xe * xe, wc, preferred_element_type=jnp.float32))
        o[...] = jax.nn.gelu(pre)

    half(xi0_ref, gs0_ref, gd0_ref, ae0_ref, oxi0)
    half(xi1_ref, gs1_ref, gd1_ref, ae1_ref, oxi1)


def _tci_call(xi0, xi1, gs0, gd0, gs1, gd1, ae0, ae1, wi_bd, wc_bd, bi_t):
    blke = lambda: pl.BlockSpec((BE, WE), lambda i: (i, 0))
    blki = lambda: pl.BlockSpec((BE, WI), lambda i: (i, 0))
    full = lambda r, c: pl.BlockSpec((r, c), lambda i: (0, 0))
    return pl.pallas_call(
        _tci_body,
        grid=(NB,),
        in_specs=([blki()] * 6 + [blke()] * 2
                  + [full(WI, WI), full(WE, WI), full(1, WI)]),
        out_specs=[blki()] * 2,
        out_shape=[jax.ShapeDtypeStruct((N_EDGES, WI), jnp.float32)] * 2,
    )(xi0, xi1, gs0, gd0, gs1, gd1, ae0, ae1, wi_bd, wc_bd, bi_t)


def _pool_body(ae0_ref, ae1_ref, xi0_ref, xi1_ref, mask_ref, o_ref):
    @pl.when(pl.program_id(0) == 0)
    def _():
        o_ref[...] = jnp.zeros_like(o_ref)

    m = mask_ref[...]
    ae = (ae0_ref[...], ae1_ref[...])
    xi = (xi0_ref[...], xi1_ref[...])
    for j in range(N_LOOPS):
        mj = m[:, j:j + 1]
        h, q = divmod(j, HALF)
        xe_j = ae[h][:, q * EQU:(q + 1) * EQU]
        xi_j = xi[h][:, q * INV:(q + 1) * INV]
        pe = jnp.sum(xe_j * mj, axis=0, keepdims=True)
        pi = jnp.sum(xi_j * mj, axis=0, keepdims=True)
        cnt = jnp.sum(mj, axis=0, keepdims=True)
        o_ref[j:j + 1, 0:EQU] += pe
        o_ref[j:j + 1, EQU:EQU + INV] += pi
        o_ref[j:j + 1, EQU + INV:EQU + INV + 1] += cnt


def _pool_call(ae0, ae1, xi0, xi1, masks_t):
    blke = pl.BlockSpec((BE, WE), lambda i: (i, 0))
    blki = pl.BlockSpec((BE, WI), lambda i: (i, 0))
    blkm = pl.BlockSpec((BE, N_LOOPS), lambda i: (i, 0))
    return pl.pallas_call(
        _pool_body,
        grid=(NB,),
        in_specs=[blke, blke, blki, blki, blkm],
        out_specs=pl.BlockSpec((N_LOOPS, HID), lambda i: (0, 0)),
        out_shape=jax.ShapeDtypeStruct((N_LOOPS, HID), jnp.float32),
    )(ae0, ae1, xi0, xi1, masks_t)


def _head_body(pr_ref, alpha_ref, h0_ref, b0_ref, h1_ref, b1_ref, h2_ref,
               b2_ref, o_ref):
    pr = pr_ref[...]
    cnt = jnp.maximum(pr[:, EQU + INV:EQU + INV + 1], 1.0)
    pooled = pr[:, :EQU + INV] / cnt
    h = jax.nn.gelu(
        jnp.dot(pooled, h0_ref[...], preferred_element_type=jnp.float32)
        + b0_ref[...])
    h = jax.nn.gelu(
        jnp.dot(h, h1_ref[...], preferred_element_type=jnp.float32)
        + b1_ref[...])
    logit = jnp.dot(h, h2_ref[...], preferred_element_type=jnp.float32) \
        + b2_ref[...]
    p = jnp.clip(jax.nn.sigmoid(logit), 1e-06, 1.0 - 1e-06)
    a = alpha_ref[...]
    lp = a * jnp.log(p) + (1.0 - a) * jnp.log(1.0 - p)
    o_ref[...] = jnp.sum(lp, axis=0, keepdims=True)


def _head_call(pr, alpha_col, h0, b0, h1, b1, h2, b2):
    return pl.pallas_call(
        _head_body,
        out_shape=jax.ShapeDtypeStruct((1, 1), jnp.float32),
    )(pr, alpha_col, h0, b0, h1, b1, h2, b2)


def kernel(alpha, sigma_seed, inv_features, edge_index, loop_indicators,
           W_equ_in, b_equ_in, W_inv_in, b_inv_in, We, Wi, bi, Wc,
           H0, b0, H1, b1, H2, b2):
    src = edge_index[0].astype(jnp.int32)
    dst = edge_index[1].astype(jnp.int32)
    masks_t = loop_indicators.T.astype(jnp.float32)
    alpha_f = alpha.astype(jnp.float32)
    zeros_e = jnp.zeros((N_NODES, WE), jnp.float32)
    zeros_i = jnp.zeros((N_NODES, WI), jnp.float32)
    eye4 = jnp.eye(HALF, dtype=jnp.float32)

    sc_e = _make_sc_e()
    sc_i = _make_sc_i()

    ae0, be0, ae1, be1, xi0, xi1 = _init_call(
        masks_t, sigma_seed.reshape(N_EDGES, 1), inv_features,
        alpha_f.reshape(1, N_LOOPS), W_equ_in.reshape(1, EQU),
        b_equ_in.reshape(1, EQU), W_inv_in, b_inv_in.reshape(1, INV))

    for l in range(N_LAYERS):
        gse0, gde0, gse1, gde1 = sc_e(ae0, be0, ae1, be1, src, dst, zeros_e)
        gsi0, gdi0, gsi1, gdi1 = sc_i(xi0, xi1, src, dst, zeros_i)
        we_bd = jnp.kron(eye4, We[l])
        wi_bd = jnp.kron(eye4, Wi[l])
        wc_bd = jnp.kron(eye4, Wc[l])
        bi_t = jnp.tile(bi[l], HALF).reshape(1, WI)
        ae0, be0, ae1, be1 = _tce_call(ae0, ae1, gse0, gde0, gse1, gde1,
                                       we_bd)
        xi0, xi1 = _tci_call(xi0, xi1, gsi0, gdi0, gsi1, gdi1, ae0, ae1,
                             wi_bd, wc_bd, bi_t)

    pr = _pool_call(ae0, ae1, xi0, xi1, masks_t)
    out = _head_call(pr, alpha_f.reshape(N_LOOPS, 1), H0,
                     b0.reshape(1, HID), H1, b1.reshape(1, HID), H2,
                     b2.reshape(1, 1))
    return out[0, 0]


# speedup vs baseline: 6.2381x; 1.0002x over previous
import functools

import jax
import jax.numpy as jnp
from jax import lax
from jax.experimental import pallas as pl
from jax.experimental.pallas import tpu as pltpu
from jax.experimental.pallas import tpu_sc as plsc

N_NODES = 10000
N_EDGES = 320000
N_LOOPS = 8
EQU = 32
INV = 16
HID = 64
N_LAYERS = 4

HALF = 4
WE = HALF * EQU
WI = HALF * INV

NS = 16
TILE_E = N_EDGES // NS
CH = 80
NCH = TILE_E // CH
NPT = N_NODES // NS

BE = 2000
NB = N_EDGES // BE

_SC_PARAMS = pltpu.CompilerParams(use_tc_tiling_on_sc=False)
_TC_PARALLEL = pltpu.CompilerParams(dimension_semantics=("parallel",))


def _sc_scratch(width):
    return [
        pltpu.VMEM_SHARED((N_NODES, width), jnp.float32),
        pltpu.VMEM((CH,), jnp.int32),
        pltpu.VMEM((CH,), jnp.int32),
        pltpu.VMEM((CH, width), jnp.float32),
        pltpu.VMEM((CH, width), jnp.float32),
    ]


def _sc_zero_node(z_h, node):
    s = lax.axis_index("s")
    nb = s * NPT
    pltpu.sync_copy(z_h.at[pl.ds(nb, NPT)], node.at[pl.ds(nb, NPT)])
    plsc.subcore_barrier()


def _sc_gather_phase(c, src_h, dst_h, node, idx_s, idx_d, abuf, bbuf,
                     gs0, gd0, gs1, gd1):
    base0 = lax.axis_index("s") * TILE_E

    def gather_half(gs, gd):
        @pl.loop(0, NCH)
        def _(kk):
            base = base0 + kk * CH
            pltpu.sync_copy(src_h.at[pl.ds(base, CH)], idx_s)
            pltpu.sync_copy(dst_h.at[pl.ds(base, CH)], idx_d)
            pltpu.sync_copy(node.at[idx_s], abuf)
            pltpu.sync_copy(node.at[idx_d], bbuf)
            pltpu.sync_copy(abuf, gs.at[pl.ds(base, CH)])
            pltpu.sync_copy(bbuf, gd.at[pl.ds(base, CH)])

    @pl.when(c == 0)
    def _():
        gather_half(gs0, gd0)

    @pl.when(c == 1)
    def _():
        gather_half(gs1, gd1)


def _make_sc_e():
    mesh = plsc.VectorSubcoreMesh(core_axis_name="c", subcore_axis_name="s")
    out_t = [jax.ShapeDtypeStruct((N_EDGES, WE), jnp.float32)] * 4

    @functools.partial(
        pl.kernel,
        out_type=out_t,
        mesh=mesh,
        compiler_params=_SC_PARAMS,
        scratch_types=_sc_scratch(WE),
    )
    def k(a0_h, b0_h, a1_h, b1_h, src_h, dst_h, z_h, gs0, gd0, gs1, gd1,
          node, idx_s, idx_d, abuf, bbuf):
        c = lax.axis_index("c")
        _sc_zero_node(z_h, node)
        base0 = lax.axis_index("s") * TILE_E

        def scatter_half(a_h, b_h):
            @pl.loop(0, NCH)
            def _(kk):
                base = base0 + kk * CH
                pltpu.sync_copy(src_h.at[pl.ds(base, CH)], idx_s)
                pltpu.sync_copy(dst_h.at[pl.ds(base, CH)], idx_d)
                pltpu.sync_copy(a_h.at[pl.ds(base, CH)], abuf)
                pltpu.sync_copy(b_h.at[pl.ds(base, CH)], bbuf)
                pltpu.sync_copy(abuf, node.at[idx_s], add=True)
                pltpu.sync_copy(bbuf, node.at[idx_d], add=True)

        @pl.when(c == 0)
        def _():
            scatter_half(a0_h, b0_h)

        @pl.when(c == 1)
        def _():
            scatter_half(a1_h, b1_h)

        plsc.subcore_barrier()
        _sc_gather_phase(c, src_h, dst_h, node, idx_s, idx_d, abuf, bbuf,
                         gs0, gd0, gs1, gd1)

    return k


def _make_sc_i():
    mesh = plsc.VectorSubcoreMesh(core_axis_name="c", subcore_axis_name="s")
    out_t = [jax.ShapeDtypeStruct((N_EDGES, WI), jnp.float32)] * 4

    @functools.partial(
        pl.kernel,
        out_type=out_t,
        mesh=mesh,
        compiler_params=_SC_PARAMS,
        scratch_types=_sc_scratch(WI),
    )
    def k(x0_h, x1_h, src_h, dst_h, z_h, gs0, gd0, gs1, gd1,
          node, idx_s, idx_d, abuf, bbuf):
        c = lax.axis_index("c")
        _sc_zero_node(z_h, node)
        base0 = lax.axis_index("s") * TILE_E

        def scatter_half(x_h):
            @pl.loop(0, NCH)
            def _(kk):
                base = base0 + kk * CH
                pltpu.sync_copy(src_h.at[pl.ds(base, CH)], idx_s)
                pltpu.sync_copy(dst_h.at[pl.ds(base, CH)], idx_d)
                pltpu.sync_copy(x_h.at[pl.ds(base, CH)], abuf)
                pltpu.sync_copy(abuf, node.at[idx_s], add=True)
                pltpu.sync_copy(abuf, node.at[idx_d], add=True)

        @pl.when(c == 0)
        def _():
            scatter_half(x0_h)

        @pl.when(c == 1)
        def _():
            scatter_half(x1_h)

        plsc.subcore_barrier()
        _sc_gather_phase(c, src_h, dst_h, node, idx_s, idx_d, abuf, bbuf,
                         gs0, gd0, gs1, gd1)

    return k


def _init_body(mask_ref, sig_ref, inv_ref, alpha_ref, wequ_ref, bequ_ref,
               winv_ref, binv_ref, oae0, obe0, oae1, obe1, oxi0, oxi1):
    m = mask_ref[...]
    afl = alpha_ref[...]
    flip = 1.0 - 2.0 * m * afl
    inv = inv_ref[...]
    xi = (inv[:, 0:1] * winv_ref[...][0:1, :]
          + inv[:, 1:2] * winv_ref[...][1:2, :]
          + binv_ref[...])
    w_equ = wequ_ref[...]
    b_equ = bequ_ref[...]
    cur = sig_ref[...]
    xes = []
    for j in range(N_LOOPS):
        xes.append(cur * w_equ + b_equ)
        cur = cur * flip[:, j:j + 1]
    oae0[...] = jnp.concatenate(xes[0:4], axis=1)
    obe0[...] = jnp.concatenate([-x for x in xes[0:4]], axis=1)
    oae1[...] = jnp.concatenate(xes[4:8], axis=1)
    obe1[...] = jnp.concatenate([-x for x in xes[4:8]], axis=1)
    oxi0[...] = jnp.concatenate([xi] * HALF, axis=1)
    oxi1[...] = jnp.concatenate([xi] * HALF, axis=1)


def _init_call(masks_t, sig_seed, inv_features, alpha_row, w_equ, b_equ,
               w_inv, b_inv):
    blk = lambda r, c: pl.BlockSpec((r, c), lambda i: (i, 0))
    full = lambda r, c: pl.BlockSpec((r, c), lambda i: (0, 0))
    oshape = ([jax.ShapeDtypeStruct((N_EDGES, WE), jnp.float32)] * 4
              + [jax.ShapeDtypeStruct((N_EDGES, WI), jnp.float32)] * 2)
    return pl.pallas_call(
        _init_body,
        grid=(NB,),
        in_specs=[blk(BE, N_LOOPS), blk(BE, 1), blk(BE, 2), full(1, N_LOOPS),
                  full(1, EQU), full(1, EQU), full(2, INV), full(1, INV)],
        out_specs=[blk(BE, WE)] * 4 + [blk(BE, WI)] * 2,
        out_shape=oshape,
        compiler_params=_TC_PARALLEL,
    )(masks_t, sig_seed, inv_features, alpha_row, w_equ, b_equ, w_inv, b_inv)


def _tce_body(ae0_ref, ae1_ref, gs0_ref, gd0_ref, gs1_ref, gd1_ref, we_ref,
              oa0, ob0, oa1, ob1):
    we = we_ref[...]

    def half(a_ref, gs_ref, gd_ref, oa, ob):
        msg = gs_ref[...] - gd_ref[...]
        xe = a_ref[...] + jnp.tanh(
            jnp.dot(msg, we, preferred_element_type=jnp.float32))
        oa[...] = xe
        ob[...] = -xe

    half(ae0_ref, gs0_ref, gd0_ref, oa0, ob0)
    half(ae1_ref, gs1_ref, gd1_ref, oa1, ob1)


def _tce_call(ae0, ae1, gs0, gd0, gs1, gd1, we_bd):
    blk = lambda: pl.BlockSpec((BE, WE), lambda i: (i, 0))
    full = pl.BlockSpec((WE, WE), lambda i: (0, 0))
    return pl.pallas_call(
        _tce_body,
        grid=(NB,),
        in_specs=[blk()] * 6 + [full],
        out_specs=[blk()] * 4,
        out_shape=[jax.ShapeDtypeStruct((N_EDGES, WE), jnp.float32)] * 4,
        compiler_params=_TC_PARALLEL,
    )(ae0, ae1, gs0, gd0, gs1, gd1, we_bd)


def _tci_body(xi0_ref, xi1_ref, gs0_ref, gd0_ref, gs1_ref, gd1_ref,
              ae0_ref, ae1_ref, wi_ref, wc_ref, bi_ref, oxi0, oxi1):
    wi = wi_ref[...]
    wc = wc_ref[...]
    bi_t = bi_ref[...]

    def half(xi_ref, gs_ref, gd_ref, ae_ref, o):
        msg = gs_ref[...] + gd_ref[...]
        xe = ae_ref[...]
        pre = (xi_ref[...]
               + jnp.dot(msg, wi, preferred_element_type=jnp.float32)
               + bi_t
               + jnp.dot(xe * xe, wc, preferred_element_type=jnp.float32))
        o[...] = jax.nn.gelu(pre)

    half(xi0_ref, gs0_ref, gd0_ref, ae0_ref, oxi0)
    half(xi1_ref, gs1_ref, gd1_ref, ae1_ref, oxi1)


def _tci_call(xi0, xi1, gs0, gd0, gs1, gd1, ae0, ae1, wi_bd, wc_bd, bi_t):
    blke = lambda: pl.BlockSpec((BE, WE), lambda i: (i, 0))
    blki = lambda: pl.BlockSpec((BE, WI), lambda i: (i, 0))
    full = lambda r, c: pl.BlockSpec((r, c), lambda i: (0, 0))
    return pl.pallas_call(
        _tci_body,
        grid=(NB,),
        in_specs=([blki()] * 6 + [blke()] * 2
                  + [full(WI, WI), full(WE, WI), full(1, WI)]),
        out_specs=[blki()] * 2,
        out_shape=[jax.ShapeDtypeStruct((N_EDGES, WI), jnp.float32)] * 2,
        compiler_params=_TC_PARALLEL,
    )(xi0, xi1, gs0, gd0, gs1, gd1, ae0, ae1, wi_bd, wc_bd, bi_t)


def _pool_body(ae0_ref, ae1_ref, xi0_ref, xi1_ref, mask_ref, o_ref):
    @pl.when(pl.program_id(0) == 0)
    def _():
        o_ref[...] = jnp.zeros_like(o_ref)

    m = mask_ref[...]
    ae = (ae0_ref[...], ae1_ref[...])
    xi = (xi0_ref[...], xi1_ref[...])
    for j in range(N_LOOPS):
        mj = m[:, j:j + 1]
        h, q = divmod(j, HALF)
        xe_j = ae[h][:, q * EQU:(q + 1) * EQU]
        xi_j = xi[h][:, q * INV:(q + 1) * INV]
        pe = jnp.sum(xe_j * mj, axis=0, keepdims=True)
        pi = jnp.sum(xi_j * mj, axis=0, keepdims=True)
        cnt = jnp.sum(mj, axis=0, keepdims=True)
        o_ref[j:j + 1, 0:EQU] += pe
        o_ref[j:j + 1, EQU:EQU + INV] += pi
        o_ref[j:j + 1, EQU + INV:EQU + INV + 1] += cnt


def _pool_call(ae0, ae1, xi0, xi1, masks_t):
    blke = pl.BlockSpec((BE, WE), lambda i: (i, 0))
    blki = pl.BlockSpec((BE, WI), lambda i: (i, 0))
    blkm = pl.BlockSpec((BE, N_LOOPS), lambda i: (i, 0))
    return pl.pallas_call(
        _pool_body,
        grid=(NB,),
        in_specs=[blke, blke, blki, blki, blkm],
        out_specs=pl.BlockSpec((N_LOOPS, HID), lambda i: (0, 0)),
        out_shape=jax.ShapeDtypeStruct((N_LOOPS, HID), jnp.float32),
    )(ae0, ae1, xi0, xi1, masks_t)


def _head_body(pr_ref, alpha_ref, h0_ref, b0_ref, h1_ref, b1_ref, h2_ref,
               b2_ref, o_ref):
    pr = pr_ref[...]
    cnt = jnp.maximum(pr[:, EQU + INV:EQU + INV + 1], 1.0)
    pooled = pr[:, :EQU + INV] / cnt
    h = jax.nn.gelu(
        jnp.dot(pooled, h0_ref[...], preferred_element_type=jnp.float32)
        + b0_ref[...])
    h = jax.nn.gelu(
        jnp.dot(h, h1_ref[...], preferred_element_type=jnp.float32)
        + b1_ref[...])
    logit = jnp.dot(h, h2_ref[...], preferred_element_type=jnp.float32) \
        + b2_ref[...]
    p = jnp.clip(jax.nn.sigmoid(logit), 1e-06, 1.0 - 1e-06)
    a = alpha_ref[...]
    lp = a * jnp.log(p) + (1.0 - a) * jnp.log(1.0 - p)
    o_ref[...] = jnp.sum(lp, axis=0, keepdims=True)


def _head_call(pr, alpha_col, h0, b0, h1, b1, h2, b2):
    return pl.pallas_call(
        _head_body,
        out_shape=jax.ShapeDtypeStruct((1, 1), jnp.float32),
    )(pr, alpha_col, h0, b0, h1, b1, h2, b2)


def kernel(alpha, sigma_seed, inv_features, edge_index, loop_indicators,
           W_equ_in, b_equ_in, W_inv_in, b_inv_in, We, Wi, bi, Wc,
           H0, b0, H1, b1, H2, b2):
    src = edge_index[0].astype(jnp.int32)
    dst = edge_index[1].astype(jnp.int32)
    masks_t = loop_indicators.T.astype(jnp.float32)
    alpha_f = alpha.astype(jnp.float32)
    zeros_e = jnp.zeros((N_NODES, WE), jnp.float32)
    zeros_i = jnp.zeros((N_NODES, WI), jnp.float32)
    eye4 = jnp.eye(HALF, dtype=jnp.float32)

    sc_e = _make_sc_e()
    sc_i = _make_sc_i()

    ae0, be0, ae1, be1, xi0, xi1 = _init_call(
        masks_t, sigma_seed.reshape(N_EDGES, 1), inv_features,
        alpha_f.reshape(1, N_LOOPS), W_equ_in.reshape(1, EQU),
        b_equ_in.reshape(1, EQU), W_inv_in, b_inv_in.reshape(1, INV))

    for l in range(N_LAYERS):
        gse0, gde0, gse1, gde1 = sc_e(ae0, be0, ae1, be1, src, dst, zeros_e)
        gsi0, gdi0, gsi1, gdi1 = sc_i(xi0, xi1, src, dst, zeros_i)
        we_bd = jnp.kron(eye4, We[l])
        wi_bd = jnp.kron(eye4, Wi[l])
        wc_bd = jnp.kron(eye4, Wc[l])
        bi_t = jnp.tile(bi[l], HALF).reshape(1, WI)
        ae0, be0, ae1, be1 = _tce_call(ae0, ae1, gse0, gde0, gse1, gde1,
                                       we_bd)
        xi0, xi1 = _tci_call(xi0, xi1, gsi0, gdi0, gsi1, gdi1, ae0, ae1,
                             wi_bd, wc_bd, bi_t)

    pr = _pool_call(ae0, ae1, xi0, xi1, masks_t)
    out = _head_call(pr, alpha_f.reshape(N_LOOPS, 1), H0,
                     b0.reshape(1, HID), H1, b1.reshape(1, HID), H2,
                     b2.reshape(1, 1))
    return out[0, 0]
